# Initial kernel scaffold; baseline (speedup 1.0000x reference)
import jax
import jax.numpy as jnp
from jax.experimental import pallas as pl


def kernel(pair_int, attribute_index, l1_table, attr_table, W, b):
    raise NotImplementedError("write your pallas kernel here")


# speedup vs baseline: 2.7002x; 2.7002x over previous
import functools

import jax
import jax.numpy as jnp
from jax import lax
from jax.experimental import pallas as pl
from jax.experimental.pallas import tpu as pltpu
from jax.experimental.pallas import tpu_sc as plsc

BATCH = 16384
DIM = 128
KOUT = 5
NPAD = 16
CROWS = 64
NWORKERS = 32
BPW = BATCH // NWORKERS
LANES = 16


def _combo_body(l1_ref, attr_ref, wt_ref, b_ref, out_ref):
    l1 = l1_ref[:]
    wt = wt_ref[:]
    bias = b_ref[:]
    for a in range(2):
        x = jnp.maximum(l1 + attr_ref[a:a + 1, :], 0.0)
        y = lax.dot_general(x, wt, (((1,), (0,)), ((), ())),
                            preferred_element_type=jnp.float32) + bias
        out_ref[a * 32:(a + 1) * 32, :] = y


_combo_call = pl.pallas_call(
    _combo_body,
    out_shape=jax.ShapeDtypeStruct((CROWS, NPAD), jnp.float32),
)


@functools.partial(
    pl.kernel,
    mesh=plsc.VectorSubcoreMesh(core_axis_name="c", subcore_axis_name="s"),
    compiler_params=pltpu.CompilerParams(needs_layout_passes=False),
    out_type=jax.ShapeDtypeStruct((BATCH * KOUT,), jnp.float32),
    scratch_types=[
        pltpu.VMEM((CROWS * NPAD,), jnp.float32),
        pltpu.VMEM((BPW,), jnp.int32),
        pltpu.VMEM((BPW,), jnp.int32),
        pltpu.VMEM((BPW * KOUT,), jnp.float32),
    ],
)
def _sc_gather(pair_hbm, attr_hbm, combo_hbm, out_hbm,
               combo_v, pair_v, attr_v, out_v):
    wid = lax.axis_index("s") * 2 + lax.axis_index("c")
    base = wid * BPW
    pltpu.sync_copy(combo_hbm, combo_v)
    pltpu.sync_copy(pair_hbm.at[pl.ds(base, BPW)], pair_v)
    pltpu.sync_copy(attr_hbm.at[pl.ds(base, BPW)], attr_v)
    for k in range(BPW // LANES):
        p = pair_v[pl.ds(k * LANES, LANES)]
        a = attr_v[pl.ds(k * LANES, LANES)]
        c = (a * 32 + p) * NPAD
        o = (lax.iota(jnp.int32, LANES) + (k * LANES)) * KOUT
        for j in range(KOUT):
            v = plsc.load_gather(combo_v, [c + j])
            plsc.store_scatter(out_v, [o + j], v)
    pltpu.sync_copy(out_v, out_hbm.at[pl.ds(base * KOUT, BPW * KOUT)])


def kernel(pair_int, attribute_index, l1_table, attr_table, W, b):
    pair_i = pair_int.astype(jnp.int32)
    attr_i = attribute_index.astype(jnp.int32)
    l1p = jnp.pad(l1_table, ((0, 32 - l1_table.shape[0]), (0, 0)))
    wt = jnp.pad(W.T, ((0, 0), (0, NPAD - KOUT)))
    b2 = jnp.pad(b, (0, NPAD - KOUT)).reshape(1, NPAD)
    combo = _combo_call(l1p, attr_table, wt, b2).reshape(CROWS * NPAD)
    flat = _sc_gather(pair_i, attr_i, combo)
    return flat.reshape(BATCH, KOUT)


# speedup vs baseline: 3.2041x; 1.1866x over previous
import functools

import jax
import jax.numpy as jnp
from jax import lax
from jax.experimental import pallas as pl
from jax.experimental.pallas import tpu as pltpu
from jax.experimental.pallas import tpu_sc as plsc

BATCH = 16384
DIM = 128
KOUT = 5
NPAD = 16
CROWS = 64
NWORKERS = 32
BPW = BATCH // NWORKERS
LANES = 16


def _combo_body(l1x_ref, attrx_ref, wt_ref, b_ref, out_ref):
    xr = jnp.maximum(l1x_ref[:] + attrx_ref[:], 0.0)
    wt = wt_ref[:]
    bias = b_ref[:]
    for q in range(8):
        y = lax.dot_general(xr[:, q * DIM:(q + 1) * DIM], wt,
                            (((1,), (0,)), ((), ())),
                            preferred_element_type=jnp.float32)
        out_ref[:, q * NPAD:(q + 1) * NPAD] = y + bias


_combo_call = pl.pallas_call(
    _combo_body,
    out_shape=jax.ShapeDtypeStruct((8, CROWS * NPAD // 8), jnp.float32),
)


@functools.partial(
    pl.kernel,
    mesh=plsc.VectorSubcoreMesh(core_axis_name="c", subcore_axis_name="s"),
    compiler_params=pltpu.CompilerParams(needs_layout_passes=False),
    out_type=jax.ShapeDtypeStruct((BATCH, KOUT), jnp.float32),
    scratch_types=[
        pltpu.VMEM((CROWS * NPAD,), jnp.float32),
        pltpu.VMEM((BPW,), jnp.int32),
        pltpu.VMEM((BPW,), jnp.int32),
        pltpu.VMEM((BPW, KOUT), jnp.float32),
    ],
)
def _sc_gather(pair_hbm, attr_hbm, combo_hbm, out_hbm,
               combo_v, pair_v, attr_v, out_v):
    wid = lax.axis_index("s") * 2 + lax.axis_index("c")
    base = wid * BPW
    pltpu.sync_copy(combo_hbm, combo_v)
    pltpu.sync_copy(pair_hbm.at[pl.ds(base, BPW)], pair_v)
    pltpu.sync_copy(attr_hbm.at[pl.ds(base, BPW)], attr_v)
    for k in range(BPW // LANES):
        p = pair_v[pl.ds(k * LANES, LANES)]
        a = attr_v[pl.ds(k * LANES, LANES)]
        c = (a * 32 + p) * NPAD
        i = lax.iota(jnp.int32, LANES) + (k * LANES)
        for j in range(KOUT):
            v = plsc.load_gather(combo_v, [c + j])
            plsc.store_scatter(out_v, [i, jnp.full((LANES,), j, jnp.int32)], v)
    pltpu.sync_copy(out_v, out_hbm.at[pl.ds(base, BPW), :])


def kernel(pair_int, attribute_index, l1_table, attr_table, W, b):
    pair_i = pair_int.astype(jnp.int32)
    attr_i = attribute_index.astype(jnp.int32)
    l1p = jnp.pad(l1_table, ((0, 32 - l1_table.shape[0]), (0, 0)))
    l1x = jnp.concatenate([l1p, l1p], axis=0).reshape(8, 8 * DIM)
    attrx = jnp.repeat(attr_table, 32, axis=0).reshape(8, 8 * DIM)
    wt = jnp.pad(W.T, ((0, 0), (0, NPAD - KOUT)))
    b2 = jnp.pad(b, (0, NPAD - KOUT)).reshape(1, NPAD)
    combo = _combo_call(l1x, attrx, wt, b2).reshape(CROWS * NPAD)
    return _sc_gather(pair_i, attr_i, combo)


# speedup vs baseline: 3.4656x; 1.0816x over previous
import functools

import jax
import jax.numpy as jnp
from jax import lax
from jax.experimental import pallas as pl
from jax.experimental.pallas import tpu as pltpu
from jax.experimental.pallas import tpu_sc as plsc

BATCH = 16384
DIM = 128
KOUT = 5
NPAD = 16
CROWS = 64
NWORKERS = 32
BPW = BATCH // NWORKERS
LANES = 16


def _combo_body(l1r_ref, attr_ref, w_ref, b_ref, out_ref):
    l1r = l1r_ref[:]
    a0 = attr_ref[0:1, :]
    a1 = attr_ref[1:2, :]
    arow0 = jnp.concatenate([a0] * 8, axis=1)
    arow1 = jnp.concatenate([a1] * 8, axis=1)
    attrx = jnp.concatenate([arow0] * 4 + [arow1] * 4, axis=0)
    l1x = jnp.concatenate([l1r, l1r], axis=0)
    xr = jnp.maximum(l1x + attrx, 0.0)
    w = jnp.concatenate(
        [w_ref[:], jnp.zeros((NPAD - KOUT, DIM), jnp.float32)], axis=0)
    bias = b_ref[:]
    for q in range(8):
        y = lax.dot_general(xr[:, q * DIM:(q + 1) * DIM], w,
                            (((1,), (1,)), ((), ())),
                            preferred_element_type=jnp.float32)
        out_ref[:, q * NPAD:(q + 1) * NPAD] = y + bias


_combo_call = pl.pallas_call(
    _combo_body,
    out_shape=jax.ShapeDtypeStruct((8, CROWS * NPAD // 8), jnp.float32),
)


@functools.partial(
    pl.kernel,
    mesh=plsc.VectorSubcoreMesh(core_axis_name="c", subcore_axis_name="s"),
    compiler_params=pltpu.CompilerParams(needs_layout_passes=False),
    out_type=jax.ShapeDtypeStruct((BATCH, KOUT), jnp.float32),
    scratch_types=[
        pltpu.VMEM((CROWS * NPAD,), jnp.float32),
        pltpu.VMEM((BPW,), jnp.int32),
        pltpu.VMEM((BPW,), jnp.int32),
        pltpu.VMEM((BPW, KOUT), jnp.float32),
    ],
)
def _sc_gather(pair_hbm, attr_hbm, combo_hbm, out_hbm,
               combo_v, pair_v, attr_v, out_v):
    wid = lax.axis_index("s") * 2 + lax.axis_index("c")
    base = wid * BPW
    pltpu.sync_copy(combo_hbm, combo_v)
    pltpu.sync_copy(pair_hbm.at[pl.ds(base, BPW)], pair_v)
    pltpu.sync_copy(attr_hbm.at[pl.ds(base, BPW)], attr_v)
    for k in range(BPW // LANES):
        p = pair_v[pl.ds(k * LANES, LANES)]
        a = attr_v[pl.ds(k * LANES, LANES)]
        c = (a * 32 + p) * NPAD
        i = lax.iota(jnp.int32, LANES) + (k * LANES)
        for j in range(KOUT):
            v = plsc.load_gather(combo_v, [c + j])
            plsc.store_scatter(out_v, [i, jnp.full((LANES,), j, jnp.int32)], v)
    pltpu.sync_copy(out_v, out_hbm.at[pl.ds(base, BPW), :])


def kernel(pair_int, attribute_index, l1_table, attr_table, W, b):
    pair_i = pair_int.astype(jnp.int32)
    attr_i = attribute_index.astype(jnp.int32)
    l1r = jnp.pad(l1_table, ((0, 32 - l1_table.shape[0]), (0, 0))).reshape(4, 8 * DIM)
    b2 = jnp.pad(b, (0, NPAD - KOUT)).reshape(1, NPAD)
    combo = _combo_call(l1r, attr_table, W, b2).reshape(CROWS * NPAD)
    return _sc_gather(pair_i, attr_i, combo)


# speedup vs baseline: 3.6599x; 1.0561x over previous
import functools

import jax
import jax.numpy as jnp
from jax import lax
from jax.experimental import pallas as pl
from jax.experimental.pallas import tpu as pltpu
from jax.experimental.pallas import tpu_sc as plsc

BATCH = 16384
DIM = 128
KOUT = 5
NPAD = 16
CROWS = 64
NWORKERS = 32
BPW = BATCH // NWORKERS
LANES = 16


def _combo_body(l1r_ref, attr_ref, w_ref, b_ref, out_ref):
    l1r = l1r_ref[:]
    a0 = attr_ref[0:1, :]
    a1 = attr_ref[1:2, :]
    arow0 = jnp.concatenate([a0] * 8, axis=1)
    arow1 = jnp.concatenate([a1] * 8, axis=1)
    attrx = jnp.concatenate([arow0] * 4 + [arow1] * 4, axis=0)
    l1x = jnp.concatenate([l1r, l1r], axis=0)
    xr = jnp.maximum(l1x + attrx, 0.0)
    w = jnp.concatenate(
        [w_ref[:], jnp.zeros((NPAD - KOUT, DIM), jnp.float32)], axis=0)
    bias = b_ref[:]
    for q in range(8):
        y = lax.dot_general(xr[:, q * DIM:(q + 1) * DIM], w,
                            (((1,), (1,)), ((), ())),
                            preferred_element_type=jnp.float32)
        out_ref[:, q * NPAD:(q + 1) * NPAD] = y + bias


_combo_call = pl.pallas_call(
    _combo_body,
    out_shape=jax.ShapeDtypeStruct((8, CROWS * NPAD // 8), jnp.float32),
)


@functools.partial(
    pl.kernel,
    mesh=plsc.VectorSubcoreMesh(core_axis_name="c", subcore_axis_name="s"),
    compiler_params=pltpu.CompilerParams(needs_layout_passes=False),
    out_type=jax.ShapeDtypeStruct((BATCH, KOUT), jnp.float32),
    scratch_types=[
        pltpu.VMEM((CROWS * NPAD,), jnp.float32),
        pltpu.VMEM((BPW,), jnp.int32),
        pltpu.VMEM((BPW,), jnp.int32),
        pltpu.VMEM((BPW, KOUT), jnp.float32),
    ],
)
def _sc_gather(pair_hbm, attr_hbm, combo_hbm, out_hbm,
               combo_v, pair_v, attr_v, out_v):
    wid = lax.axis_index("s") * 2 + lax.axis_index("c")
    base = wid * BPW
    pltpu.sync_copy(combo_hbm, combo_v)
    pltpu.sync_copy(pair_hbm.at[pl.ds(base, BPW)], pair_v)
    pltpu.sync_copy(attr_hbm.at[pl.ds(base, BPW)], attr_v)
    @plsc.parallel_loop(0, BPW // LANES, 1, unroll=4)
    def _chunk(k):
        kl = k * LANES
        p = pair_v[pl.ds(kl, LANES)]
        a = attr_v[pl.ds(kl, LANES)]
        c = (a * 32 + p) * NPAD
        i = lax.iota(jnp.int32, LANES) + kl
        for j in range(KOUT):
            v = plsc.load_gather(combo_v, [c + j])
            plsc.store_scatter(out_v, [i, jnp.full((LANES,), j, jnp.int32)], v)
    pltpu.sync_copy(out_v, out_hbm.at[pl.ds(base, BPW), :])


def kernel(pair_int, attribute_index, l1_table, attr_table, W, b):
    pair_i = pair_int.astype(jnp.int32)
    attr_i = attribute_index.astype(jnp.int32)
    l1r = jnp.pad(l1_table, ((0, 32 - l1_table.shape[0]), (0, 0))).reshape(4, 8 * DIM)
    b2 = jnp.pad(b, (0, NPAD - KOUT)).reshape(1, NPAD)
    combo = _combo_call(l1r, attr_table, W, b2).reshape(CROWS * NPAD)
    return _sc_gather(pair_i, attr_i, combo)


# speedup vs baseline: 3.6638x; 1.0011x over previous
import functools

import jax
import jax.numpy as jnp
from jax import lax
from jax.experimental import pallas as pl
from jax.experimental.pallas import tpu as pltpu
from jax.experimental.pallas import tpu_sc as plsc

BATCH = 16384
DIM = 128
KOUT = 5
NPAD = 16
CROWS = 64
NWORKERS = 32
BPW = BATCH // NWORKERS
LANES = 16


def _combo_body(l1r_ref, attr_ref, w_ref, b_ref, out_ref):
    l1r = l1r_ref[:]
    a0 = attr_ref[0:1, :]
    a1 = attr_ref[1:2, :]
    arow0 = jnp.concatenate([a0] * 8, axis=1)
    arow1 = jnp.concatenate([a1] * 8, axis=1)
    attrx = jnp.concatenate([arow0] * 4 + [arow1] * 4, axis=0)
    l1x = jnp.concatenate([l1r, l1r], axis=0)
    xr = jnp.maximum(l1x + attrx, 0.0)
    w = jnp.concatenate(
        [w_ref[:], jnp.zeros((NPAD - KOUT, DIM), jnp.float32)], axis=0)
    bias = b_ref[:]
    for q in range(8):
        y = lax.dot_general(xr[:, q * DIM:(q + 1) * DIM], w,
                            (((1,), (1,)), ((), ())),
                            preferred_element_type=jnp.float32)
        out_ref[:, q * NPAD:(q + 1) * NPAD] = y + bias


_combo_call = pl.pallas_call(
    _combo_body,
    out_shape=jax.ShapeDtypeStruct((8, CROWS * NPAD // 8), jnp.float32),
)


@functools.partial(
    pl.kernel,
    mesh=plsc.VectorSubcoreMesh(core_axis_name="c", subcore_axis_name="s"),
    compiler_params=pltpu.CompilerParams(needs_layout_passes=False),
    out_type=jax.ShapeDtypeStruct((BATCH, KOUT), jnp.float32),
    scratch_types=[
        pltpu.VMEM((CROWS * NPAD,), jnp.float32),
        pltpu.VMEM((BPW,), jnp.int32),
        pltpu.VMEM((BPW,), jnp.int32),
        pltpu.VMEM((BPW, KOUT), jnp.float32),
    ],
)
def _sc_gather(pair_hbm, attr_hbm, combo_hbm, out_hbm,
               combo_v, pair_v, attr_v, out_v):
    wid = lax.axis_index("s") * 2 + lax.axis_index("c")
    base = wid * BPW
    pltpu.sync_copy(combo_hbm, combo_v)
    pltpu.sync_copy(pair_hbm.at[pl.ds(base, BPW)], pair_v)
    pltpu.sync_copy(attr_hbm.at[pl.ds(base, BPW)], attr_v)
    @plsc.parallel_loop(0, BPW // LANES, 1, unroll=2)
    def _chunk(k):
        kl = k * LANES
        p = pair_v[pl.ds(kl, LANES)]
        a = attr_v[pl.ds(kl, LANES)]
        c = (a * 32 + p) * NPAD
        i = lax.iota(jnp.int32, LANES) + kl
        for j in range(KOUT):
            v = plsc.load_gather(combo_v, [c + j])
            plsc.store_scatter(out_v, [i, jnp.full((LANES,), j, jnp.int32)], v)
    pltpu.sync_copy(out_v, out_hbm.at[pl.ds(base, BPW), :])


def kernel(pair_int, attribute_index, l1_table, attr_table, W, b):
    pair_i = pair_int.astype(jnp.int32)
    attr_i = attribute_index.astype(jnp.int32)
    l1r = jnp.pad(l1_table, ((0, 32 - l1_table.shape[0]), (0, 0))).reshape(4, 8 * DIM)
    b2 = jnp.pad(b, (0, NPAD - KOUT)).reshape(1, NPAD)
    combo = _combo_call(l1r, attr_table, W, b2).reshape(CROWS * NPAD)
    return _sc_gather(pair_i, attr_i, combo)


# speedup vs baseline: 3.9605x; 1.0810x over previous
import functools

import jax
import jax.numpy as jnp
from jax import lax
from jax.experimental import pallas as pl
from jax.experimental.pallas import tpu as pltpu
from jax.experimental.pallas import tpu_sc as plsc

BATCH = 16384
DIM = 128
KOUT = 5
NPAD = 16
NWORKERS = 32
BPW = BATCH // NWORKERS
LANES = 16


def _combo_body(l1_ref, attr_ref, w_ref, b_ref, out_ref):
    l1 = jnp.concatenate(
        [l1_ref[:], jnp.zeros((32 - l1_ref.shape[0], DIM), jnp.float32)], axis=0)
    w = jnp.concatenate(
        [w_ref[:], jnp.zeros((NPAD - KOUT, DIM), jnp.float32)], axis=0)
    bias = jnp.concatenate(
        [b_ref[:], jnp.zeros((1, NPAD - KOUT), jnp.float32)], axis=1)
    for a in range(2):
        x = jnp.maximum(l1 + attr_ref[a:a + 1, :], 0.0)
        y = lax.dot_general(x, w, (((1,), (1,)), ((), ())),
                            preferred_element_type=jnp.float32) + bias
        for s in range(4):
            row = jnp.concatenate(
                [y[8 * s + q:8 * s + q + 1, :] for q in range(8)], axis=1)
            out_ref[a * 4 + s:a * 4 + s + 1, :] = row


_combo_call = pl.pallas_call(
    _combo_body,
    out_shape=jax.ShapeDtypeStruct((8, 128), jnp.float32),
)


@functools.partial(
    pl.kernel,
    mesh=plsc.VectorSubcoreMesh(core_axis_name="c", subcore_axis_name="s"),
    compiler_params=pltpu.CompilerParams(needs_layout_passes=False),
    out_type=jax.ShapeDtypeStruct((BATCH, KOUT), jnp.float32),
    scratch_types=[
        pltpu.VMEM((8, 128), jnp.float32),
        pltpu.VMEM((BPW,), jnp.int32),
        pltpu.VMEM((BPW,), jnp.int32),
        pltpu.VMEM((BPW, KOUT), jnp.float32),
        pltpu.SemaphoreType.DMA,
    ],
)
def _sc_gather(pair_hbm, attr_hbm, combo_hbm, out_hbm,
               combo_v, pair_v, attr_v, out_v, sem):
    wid = lax.axis_index("s") * 2 + lax.axis_index("c")
    base = wid * BPW
    d0 = pltpu.async_copy(combo_hbm, combo_v, sem)
    d1 = pltpu.async_copy(pair_hbm.at[pl.ds(base, BPW)], pair_v, sem)
    d2 = pltpu.async_copy(attr_hbm.at[pl.ds(base, BPW)], attr_v, sem)
    d0.wait()
    d1.wait()
    d2.wait()

    @plsc.parallel_loop(0, BPW // LANES, 1, unroll=4)
    def _chunk(k):
        kl = k * LANES
        p = pair_v[pl.ds(kl, LANES)]
        a = attr_v[pl.ds(kl, LANES)]
        c = (a * 32 + p) * NPAD
        i = lax.iota(jnp.int32, LANES) + kl
        for j in range(KOUT):
            t = c + j
            v = plsc.load_gather(
                combo_v, [lax.shift_right_logical(t, 7), t & 127])
            plsc.store_scatter(out_v, [i, jnp.full((LANES,), j, jnp.int32)], v)

    pltpu.sync_copy(out_v, out_hbm.at[pl.ds(base, BPW), :])


def kernel(pair_int, attribute_index, l1_table, attr_table, W, b):
    pair_i = pair_int.astype(jnp.int32)
    attr_i = attribute_index.astype(jnp.int32)
    combo = _combo_call(l1_table, attr_table, W, b.reshape(1, KOUT))
    return _sc_gather(pair_i, attr_i, combo)


# speedup vs baseline: 3.9630x; 1.0006x over previous
import functools

import jax
import jax.numpy as jnp
from jax import lax
from jax.experimental import pallas as pl
from jax.experimental.pallas import tpu as pltpu
from jax.experimental.pallas import tpu_sc as plsc

BATCH = 16384
DIM = 128
KOUT = 5
NPAD = 16
NWORKERS = 32
BPW = BATCH // NWORKERS
LANES = 16


def _combo_body(l1_ref, attr_ref, w_ref, b_ref, out_ref):
    l1 = jnp.concatenate(
        [l1_ref[:], jnp.zeros((32 - l1_ref.shape[0], DIM), jnp.float32)], axis=0)
    w = jnp.concatenate(
        [w_ref[:], jnp.zeros((NPAD - KOUT, DIM), jnp.float32)], axis=0)
    bias = jnp.concatenate(
        [b_ref[:], jnp.zeros((1, NPAD - KOUT), jnp.float32)], axis=1)
    for a in range(2):
        x = jnp.maximum(l1 + attr_ref[a:a + 1, :], 0.0)
        y = lax.dot_general(x, w, (((1,), (1,)), ((), ())),
                            preferred_element_type=jnp.float32) + bias
        for s in range(4):
            row = jnp.concatenate(
                [y[8 * s + q:8 * s + q + 1, :] for q in range(8)], axis=1)
            out_ref[a * 4 + s:a * 4 + s + 1, :] = row


_combo_call = pl.pallas_call(
    _combo_body,
    out_shape=jax.ShapeDtypeStruct((8, 128), jnp.float32),
)


@functools.partial(
    pl.kernel,
    mesh=plsc.VectorSubcoreMesh(core_axis_name="c", subcore_axis_name="s"),
    compiler_params=pltpu.CompilerParams(needs_layout_passes=False, skip_device_barrier=True),
    out_type=jax.ShapeDtypeStruct((BATCH, KOUT), jnp.float32),
    scratch_types=[
        pltpu.VMEM((8, 128), jnp.float32),
        pltpu.VMEM((BPW,), jnp.int32),
        pltpu.VMEM((BPW,), jnp.int32),
        pltpu.VMEM((BPW, KOUT), jnp.float32),
        pltpu.SemaphoreType.DMA,
    ],
)
def _sc_gather(pair_hbm, attr_hbm, combo_hbm, out_hbm,
               combo_v, pair_v, attr_v, out_v, sem):
    wid = lax.axis_index("s") * 2 + lax.axis_index("c")
    base = wid * BPW
    d0 = pltpu.async_copy(combo_hbm, combo_v, sem)
    d1 = pltpu.async_copy(pair_hbm.at[pl.ds(base, BPW)], pair_v, sem)
    d2 = pltpu.async_copy(attr_hbm.at[pl.ds(base, BPW)], attr_v, sem)
    d0.wait()
    d1.wait()
    d2.wait()

    @plsc.parallel_loop(0, BPW // LANES, 1, unroll=4)
    def _chunk(k):
        kl = k * LANES
        p = pair_v[pl.ds(kl, LANES)]
        a = attr_v[pl.ds(kl, LANES)]
        c = (a * 32 + p) * NPAD
        i = lax.iota(jnp.int32, LANES) + kl
        for j in range(KOUT):
            t = c + j
            v = plsc.load_gather(
                combo_v, [lax.shift_right_logical(t, 7), t & 127])
            plsc.store_scatter(out_v, [i, jnp.full((LANES,), j, jnp.int32)], v)

    pltpu.sync_copy(out_v, out_hbm.at[pl.ds(base, BPW), :])


def kernel(pair_int, attribute_index, l1_table, attr_table, W, b):
    pair_i = pair_int.astype(jnp.int32)
    attr_i = attribute_index.astype(jnp.int32)
    combo = _combo_call(l1_table, attr_table, W, b.reshape(1, KOUT))
    return _sc_gather(pair_i, attr_i, combo)
